# baseline (speedup 1.0000x reference)
import functools

import jax
import jax.numpy as jnp
from jax import lax
from jax.experimental import pallas as pl
from jax.experimental.pallas import tpu as pltpu
from jax.experimental.pallas import tpu_sc as plsc

H = 768
F = 3072
E = 8
K = 2
T = 2048
A = T * K
TILE = 512
NT = (A + E * TILE) // TILE
A_PAD = NT * TILE
FC = 1536
NF = F // FC
RT = 256
AUX_COEF = 0.01

NW = 32
TPW = T // NW
CH = 64
GR = 16
GC = T // GR


def _router_body(x_ref, g_ref, b_ref, gw_ref,
                 normed_ref, isel_ref, wsel_ref, psum_ref, csum_ref, aux_ref):
    i = pl.program_id(0)
    nsteps = pl.num_programs(0)

    @pl.when(i == 0)
    def _init():
        psum_ref[...] = jnp.zeros_like(psum_ref)
        csum_ref[...] = jnp.zeros_like(csum_ref)

    x = x_ref[...]
    mu = jnp.mean(x, axis=-1, keepdims=True)
    xc = x - mu
    var = jnp.mean(xc * xc, axis=-1, keepdims=True)
    normed = xc * jax.lax.rsqrt(var + 1e-5) * g_ref[...] + b_ref[...]
    normed_ref[...] = normed

    logits = jax.lax.dot_general(normed, gw_ref[...],
                                 (((1,), (1,)), ((), ())),
                                 preferred_element_type=jnp.float32)
    eidx = jax.lax.broadcasted_iota(jnp.int32, logits.shape, 1)
    m1 = jnp.max(logits, axis=-1, keepdims=True)
    i1 = jnp.min(jnp.where(logits == m1, eidx, E), axis=-1, keepdims=True)
    logits2 = jnp.where(eidx == i1, -jnp.inf, logits)
    m2 = jnp.max(logits2, axis=-1, keepdims=True)
    i2 = jnp.min(jnp.where(logits2 == m2, eidx, E), axis=-1, keepdims=True)
    e2 = jnp.exp(m2 - m1)
    w1 = 1.0 / (1.0 + e2)
    w2 = e2 * w1
    isel_ref[...] = jnp.concatenate([i1, i2], axis=1)
    wsel_ref[...] = jnp.concatenate([w1, w2], axis=1)

    ex = jnp.exp(logits - m1)
    probs = ex / jnp.sum(ex, axis=-1, keepdims=True)
    psum_ref[...] += jnp.sum(probs, axis=0, keepdims=True)
    sel = jnp.logical_or(eidx == i1, eidx == i2).astype(jnp.float32)
    csum_ref[...] += jnp.sum(sel, axis=0, keepdims=True)

    @pl.when(i == nsteps - 1)
    def _fin():
        frac = csum_ref[...] / T
        pmean = psum_ref[...] / T
        aux_ref[...] = (AUX_COEF * E * jnp.sum(frac * pmean)).reshape(1, 1)


def _router(x, gate_W, ln_g, ln_b):
    return pl.pallas_call(
        _router_body,
        grid=(T // RT,),
        in_specs=[
            pl.BlockSpec((RT, H), lambda i: (i, 0)),
            pl.BlockSpec((1, H), lambda i: (0, 0)),
            pl.BlockSpec((1, H), lambda i: (0, 0)),
            pl.BlockSpec((E, H), lambda i: (0, 0)),
        ],
        out_specs=[
            pl.BlockSpec((RT, H), lambda i: (i, 0)),
            pl.BlockSpec((RT, K), lambda i: (i, 0)),
            pl.BlockSpec((RT, K), lambda i: (i, 0)),
            pl.BlockSpec((1, E), lambda i: (0, 0)),
            pl.BlockSpec((1, E), lambda i: (0, 0)),
            pl.BlockSpec((1, 1), lambda i: (0, 0)),
        ],
        out_shape=[
            jax.ShapeDtypeStruct((T, H), jnp.float32),
            jax.ShapeDtypeStruct((T, K), jnp.int32),
            jax.ShapeDtypeStruct((T, K), jnp.float32),
            jax.ShapeDtypeStruct((1, E), jnp.float32),
            jax.ShapeDtypeStruct((1, E), jnp.float32),
            jax.ShapeDtypeStruct((1, 1), jnp.float32),
        ],
        compiler_params=pltpu.CompilerParams(
            dimension_semantics=("arbitrary",)),
    )(x, ln_g.reshape(1, H), ln_b.reshape(1, H), gate_W)


def _glue_body(e0_ref, e1_ref, p0_ref, p1_ref, te_ref, tv_ref):
    e0 = e0_ref[...]
    e1 = e1_ref[...]
    lane = jax.lax.broadcasted_iota(jnp.int32, (GR, GC), 1)
    sub = jax.lax.broadcasted_iota(jnp.int32, (GR, 1), 0)

    counts = []
    prefs = []
    for e in range(E):
        m = (e0 == e).astype(jnp.int32) + (e1 == e).astype(jnp.int32)
        c = m
        s = 1
        while s < GC:
            c = c + jnp.where(lane >= s, pltpu.roll(c, s, axis=1), 0)
            s *= 2
        rowtot = jnp.sum(m, axis=1, keepdims=True)
        rp = rowtot
        s = 1
        while s < GR:
            rp = rp + jnp.where(sub >= s, pltpu.roll(rp, s, axis=0), 0)
            s *= 2
        excl = (rp - rowtot) + (c - m)
        counts.append(jnp.sum(m))
        prefs.append(excl)

    gs = []
    total_tiles = jnp.int32(0)
    cum_tiles = []
    for e in range(E):
        tiles_e = (counts[e] + (TILE - 1)) // TILE
        gs.append(total_tiles * TILE)
        total_tiles = total_tiles + tiles_e
        cum_tiles.append(total_tiles)

    pos0 = jnp.zeros((GR, GC), jnp.int32)
    pos1 = jnp.zeros((GR, GC), jnp.int32)
    for e in range(E):
        pos0 = jnp.where(e0 == e, gs[e] + prefs[e], pos0)
        pos1 = jnp.where(e1 == e, gs[e] + prefs[e], pos1)
    p0_ref[...] = pos0
    p1_ref[...] = pos1

    tlane = jax.lax.broadcasted_iota(jnp.int32, (1, NT), 1)
    tev = jnp.zeros((1, NT), jnp.int32)
    for e in range(E):
        tev = tev + (tlane >= cum_tiles[e]).astype(jnp.int32)
    te_ref[...] = jnp.minimum(tev, E - 1)
    tv_ref[...] = (tlane < total_tiles).astype(jnp.int32)


def _glue(e0_2d, e1_2d):
    return pl.pallas_call(
        _glue_body,
        grid=(1,),
        in_specs=[pl.BlockSpec((GR, GC), lambda i: (0, 0))] * 2,
        out_specs=[
            pl.BlockSpec((GR, GC), lambda i: (0, 0)),
            pl.BlockSpec((GR, GC), lambda i: (0, 0)),
            pl.BlockSpec((1, NT), lambda i: (0, 0)),
            pl.BlockSpec((1, NT), lambda i: (0, 0)),
        ],
        out_shape=[
            jax.ShapeDtypeStruct((GR, GC), jnp.int32),
            jax.ShapeDtypeStruct((GR, GC), jnp.int32),
            jax.ShapeDtypeStruct((1, NT), jnp.int32),
            jax.ShapeDtypeStruct((1, NT), jnp.int32),
        ],
    )(e0_2d, e1_2d)


@functools.partial(
    pl.kernel,
    out_type=jax.ShapeDtypeStruct((A_PAD, H), jnp.float32),
    mesh=plsc.VectorSubcoreMesh(core_axis_name="c", subcore_axis_name="s"),
    scratch_types=[
        pltpu.VMEM((K, TPW), jnp.int32),
        pltpu.VMEM((TPW, H), jnp.float32),
        pltpu.SemaphoreType.DMA,
        pltpu.SemaphoreType.DMA,
    ],
)
def _dispatch_scatter(normed_hbm, idx3_hbm, xs_hbm, idx_v, buf, sem0, sem1):
    wid = lax.axis_index("s") * 2 + lax.axis_index("c")
    pltpu.sync_copy(idx3_hbm.at[wid], idx_v)
    pltpu.sync_copy(normed_hbm.at[pl.ds(wid * TPW, TPW)], buf)
    s0 = pltpu.async_copy(buf, xs_hbm.at[idx_v.at[0]], sem0)
    s1 = pltpu.async_copy(buf, xs_hbm.at[idx_v.at[1]], sem1)
    s0.wait()
    s1.wait()


@functools.partial(
    pl.kernel,
    out_type=(jax.ShapeDtypeStruct((T, H), jnp.float32),
              jax.ShapeDtypeStruct((T, H), jnp.float32)),
    mesh=plsc.VectorSubcoreMesh(core_axis_name="c", subcore_axis_name="s"),
    scratch_types=[
        pltpu.VMEM((K, TPW), jnp.int32),
        pltpu.VMEM((TPW, H), jnp.float32),
        pltpu.VMEM((TPW, H), jnp.float32),
        pltpu.SemaphoreType.DMA,
        pltpu.SemaphoreType.DMA,
    ],
)
def _combine_gather(y_hbm, idx3_hbm, y0_hbm, y1_hbm, idx_v, b0, b1,
                    sem0, sem1):
    wid = lax.axis_index("s") * 2 + lax.axis_index("c")
    base = wid * TPW
    pltpu.sync_copy(idx3_hbm.at[wid], idx_v)
    g0 = pltpu.async_copy(y_hbm.at[idx_v.at[0]], b0, sem0)
    g1 = pltpu.async_copy(y_hbm.at[idx_v.at[1]], b1, sem1)
    g0.wait()
    s0 = pltpu.async_copy(b0, y0_hbm.at[pl.ds(base, TPW)], sem0)
    g1.wait()
    s1 = pltpu.async_copy(b1, y1_hbm.at[pl.ds(base, TPW)], sem1)
    s0.wait()
    s1.wait()


def _ffn_body(te_ref, tv_ref, xs_ref, wg_ref, wu_ref, wd_ref, y_ref):
    i = pl.program_id(0)
    j = pl.program_id(1)

    @pl.when(tv_ref[i] == 1)
    def _():
        x = xs_ref[...].astype(jnp.bfloat16)
        g = jnp.dot(x, wg_ref[0], preferred_element_type=jnp.float32)
        u = jnp.dot(x, wu_ref[0], preferred_element_type=jnp.float32)
        h = ((g * jax.nn.sigmoid(g)) * u).astype(jnp.bfloat16)
        part = jnp.dot(h, wd_ref[0], preferred_element_type=jnp.float32)

        @pl.when(j == 0)
        def _a():
            y_ref[...] = part

        @pl.when(j > 0)
        def _b():
            y_ref[...] += part


def _grouped_ffn(xs, W_gate, W_up, W_down, tile_expert, tile_valid):
    grid_spec = pltpu.PrefetchScalarGridSpec(
        num_scalar_prefetch=2,
        grid=(NT, NF),
        in_specs=[
            pl.BlockSpec((TILE, H), lambda i, j, te, tv: (i, 0)),
            pl.BlockSpec((1, H, FC), lambda i, j, te, tv: (te[i], 0, j)),
            pl.BlockSpec((1, H, FC), lambda i, j, te, tv: (te[i], 0, j)),
            pl.BlockSpec((1, FC, H), lambda i, j, te, tv: (te[i], j, 0)),
        ],
        out_specs=pl.BlockSpec((TILE, H), lambda i, j, te, tv: (i, 0)),
    )
    return pl.pallas_call(
        _ffn_body,
        grid_spec=grid_spec,
        out_shape=jax.ShapeDtypeStruct((A_PAD, H), jnp.float32),
        compiler_params=pltpu.CompilerParams(
            dimension_semantics=("arbitrary", "arbitrary")),
    )(tile_expert, tile_valid, xs, W_gate, W_up, W_down)


def _combine_body(x_ref, y0_ref, y1_ref, w_ref, out_ref):
    w = w_ref[...]
    out_ref[...] = (x_ref[...] + w[:, 0:1] * y0_ref[...]
                    + w[:, 1:2] * y1_ref[...])


def _combine(x, y0, y1, wsel):
    return pl.pallas_call(
        _combine_body,
        grid=(T // RT,),
        in_specs=[
            pl.BlockSpec((RT, H), lambda i: (i, 0)),
            pl.BlockSpec((RT, H), lambda i: (i, 0)),
            pl.BlockSpec((RT, H), lambda i: (i, 0)),
            pl.BlockSpec((RT, K), lambda i: (i, 0)),
        ],
        out_specs=pl.BlockSpec((RT, H), lambda i: (i, 0)),
        out_shape=jax.ShapeDtypeStruct((T, H), jnp.float32),
    )(x, y0, y1, wsel)


def kernel(hidden_states, gate_W, W_gate, W_up, W_down, ln_g, ln_b):
    B, S, _ = hidden_states.shape
    x = hidden_states.reshape(T, H)

    normed, isel, wsel, _, _, aux = _router(x, gate_W, ln_g, ln_b)

    e0_2d = isel[:, 0].reshape(GR, GC)
    e1_2d = isel[:, 1].reshape(GR, GC)
    p0_2d, p1_2d, te_2d, tv_2d = _glue(e0_2d, e1_2d)

    idx3 = jnp.stack([p0_2d.reshape(NW, TPW), p1_2d.reshape(NW, TPW)],
                     axis=1)
    tile_expert = te_2d.reshape(NT)
    tile_valid = tv_2d.reshape(NT)

    xs = _dispatch_scatter(normed, idx3)

    y = _grouped_ffn(xs, W_gate.astype(jnp.bfloat16),
                     W_up.astype(jnp.bfloat16), W_down.astype(jnp.bfloat16),
                     tile_expert, tile_valid)

    y0, y1 = _combine_gather(y, idx3)
    out = _combine(x, y0, y1, wsel)

    return out.reshape(B, S, H), aux[0, 0]


# speedup vs baseline: 1.3197x; 1.3197x over previous
import functools

import jax
import jax.numpy as jnp
from jax import lax
from jax.experimental import pallas as pl
from jax.experimental.pallas import tpu as pltpu
from jax.experimental.pallas import tpu_sc as plsc

H = 768
F = 3072
E = 8
K = 2
T = 2048
A = T * K
TILE = 512
NT = (A + E * TILE) // TILE
A_PAD = NT * TILE
FC = 1536
NF = F // FC
RT = 256
AUX_COEF = 0.01

NW = 32
TPW = T // NW
CH = 64
GR = 16
GC = T // GR


def _router_body(x_ref, g_ref, b_ref, gw_ref,
                 normed_ref, isel_ref, wsel_ref, psum_ref, csum_ref, aux_ref):
    i = pl.program_id(0)
    nsteps = pl.num_programs(0)

    @pl.when(i == 0)
    def _init():
        psum_ref[...] = jnp.zeros_like(psum_ref)
        csum_ref[...] = jnp.zeros_like(csum_ref)

    x = x_ref[...]
    mu = jnp.mean(x, axis=-1, keepdims=True)
    xc = x - mu
    var = jnp.mean(xc * xc, axis=-1, keepdims=True)
    normed = xc * jax.lax.rsqrt(var + 1e-5) * g_ref[...] + b_ref[...]
    normed_ref[...] = normed

    logits = jax.lax.dot_general(normed, gw_ref[...],
                                 (((1,), (1,)), ((), ())),
                                 preferred_element_type=jnp.float32)
    eidx = jax.lax.broadcasted_iota(jnp.int32, logits.shape, 1)
    m1 = jnp.max(logits, axis=-1, keepdims=True)
    i1 = jnp.min(jnp.where(logits == m1, eidx, E), axis=-1, keepdims=True)
    logits2 = jnp.where(eidx == i1, -jnp.inf, logits)
    m2 = jnp.max(logits2, axis=-1, keepdims=True)
    i2 = jnp.min(jnp.where(logits2 == m2, eidx, E), axis=-1, keepdims=True)
    e2 = jnp.exp(m2 - m1)
    w1 = 1.0 / (1.0 + e2)
    w2 = e2 * w1
    isel_ref[...] = jnp.concatenate([i1, i2], axis=1)
    wsel_ref[...] = jnp.concatenate([w1, w2], axis=1)

    ex = jnp.exp(logits - m1)
    probs = ex / jnp.sum(ex, axis=-1, keepdims=True)
    psum_ref[...] += jnp.sum(probs, axis=0, keepdims=True)
    sel = jnp.logical_or(eidx == i1, eidx == i2).astype(jnp.float32)
    csum_ref[...] += jnp.sum(sel, axis=0, keepdims=True)

    @pl.when(i == nsteps - 1)
    def _fin():
        frac = csum_ref[...] / T
        pmean = psum_ref[...] / T
        aux_ref[...] = (AUX_COEF * E * jnp.sum(frac * pmean)).reshape(1, 1)


def _router(x, gate_W, ln_g, ln_b):
    return pl.pallas_call(
        _router_body,
        grid=(T // RT,),
        in_specs=[
            pl.BlockSpec((RT, H), lambda i: (i, 0)),
            pl.BlockSpec((1, H), lambda i: (0, 0)),
            pl.BlockSpec((1, H), lambda i: (0, 0)),
            pl.BlockSpec((E, H), lambda i: (0, 0)),
        ],
        out_specs=[
            pl.BlockSpec((RT, H), lambda i: (i, 0)),
            pl.BlockSpec((RT, K), lambda i: (i, 0)),
            pl.BlockSpec((RT, K), lambda i: (i, 0)),
            pl.BlockSpec((1, E), lambda i: (0, 0)),
            pl.BlockSpec((1, E), lambda i: (0, 0)),
            pl.BlockSpec((1, 1), lambda i: (0, 0)),
        ],
        out_shape=[
            jax.ShapeDtypeStruct((T, H), jnp.float32),
            jax.ShapeDtypeStruct((T, K), jnp.int32),
            jax.ShapeDtypeStruct((T, K), jnp.float32),
            jax.ShapeDtypeStruct((1, E), jnp.float32),
            jax.ShapeDtypeStruct((1, E), jnp.float32),
            jax.ShapeDtypeStruct((1, 1), jnp.float32),
        ],
        compiler_params=pltpu.CompilerParams(
            dimension_semantics=("arbitrary",)),
    )(x, ln_g.reshape(1, H), ln_b.reshape(1, H), gate_W)


def _glue_body(e0_ref, e1_ref, p0_ref, p1_ref, te_ref, tv_ref):
    e0 = e0_ref[...]
    e1 = e1_ref[...]
    lane = jax.lax.broadcasted_iota(jnp.int32, (GR, GC), 1)
    sub = jax.lax.broadcasted_iota(jnp.int32, (GR, 1), 0)

    counts = []
    prefs = []
    for e in range(E):
        m = (e0 == e).astype(jnp.int32) + (e1 == e).astype(jnp.int32)
        c = m
        s = 1
        while s < GC:
            c = c + jnp.where(lane >= s, pltpu.roll(c, s, axis=1), 0)
            s *= 2
        rowtot = jnp.sum(m, axis=1, keepdims=True)
        rp = rowtot
        s = 1
        while s < GR:
            rp = rp + jnp.where(sub >= s, pltpu.roll(rp, s, axis=0), 0)
            s *= 2
        excl = (rp - rowtot) + (c - m)
        counts.append(jnp.sum(m))
        prefs.append(excl)

    gs = []
    total_tiles = jnp.int32(0)
    cum_tiles = []
    for e in range(E):
        tiles_e = (counts[e] + (TILE - 1)) // TILE
        gs.append(total_tiles * TILE)
        total_tiles = total_tiles + tiles_e
        cum_tiles.append(total_tiles)

    pos0 = jnp.zeros((GR, GC), jnp.int32)
    pos1 = jnp.zeros((GR, GC), jnp.int32)
    for e in range(E):
        pos0 = jnp.where(e0 == e, gs[e] + prefs[e], pos0)
        pos1 = jnp.where(e1 == e, gs[e] + prefs[e], pos1)
    p0_ref[...] = pos0
    p1_ref[...] = pos1

    tlane = jax.lax.broadcasted_iota(jnp.int32, (1, NT), 1)
    tev = jnp.zeros((1, NT), jnp.int32)
    for e in range(E):
        tev = tev + (tlane >= cum_tiles[e]).astype(jnp.int32)
    te_ref[...] = jnp.minimum(tev, E - 1)
    tv_ref[...] = (tlane < total_tiles).astype(jnp.int32)


def _glue(e0_2d, e1_2d):
    return pl.pallas_call(
        _glue_body,
        grid=(1,),
        in_specs=[pl.BlockSpec((GR, GC), lambda i: (0, 0))] * 2,
        out_specs=[
            pl.BlockSpec((GR, GC), lambda i: (0, 0)),
            pl.BlockSpec((GR, GC), lambda i: (0, 0)),
            pl.BlockSpec((1, NT), lambda i: (0, 0)),
            pl.BlockSpec((1, NT), lambda i: (0, 0)),
        ],
        out_shape=[
            jax.ShapeDtypeStruct((GR, GC), jnp.int32),
            jax.ShapeDtypeStruct((GR, GC), jnp.int32),
            jax.ShapeDtypeStruct((1, NT), jnp.int32),
            jax.ShapeDtypeStruct((1, NT), jnp.int32),
        ],
    )(e0_2d, e1_2d)


@functools.partial(
    pl.kernel,
    out_type=jax.ShapeDtypeStruct((A_PAD, H), jnp.float32),
    mesh=plsc.VectorSubcoreMesh(core_axis_name="c", subcore_axis_name="s"),
    scratch_types=[
        pltpu.VMEM((K, TPW), jnp.int32),
        pltpu.VMEM((TPW, H), jnp.float32),
        pltpu.SemaphoreType.DMA,
        pltpu.SemaphoreType.DMA,
    ],
)
def _dispatch_scatter(normed_hbm, idx3_hbm, xs_hbm, idx_v, buf, sem0, sem1):
    wid = lax.axis_index("s") * 2 + lax.axis_index("c")
    pltpu.sync_copy(idx3_hbm.at[wid], idx_v)
    pltpu.sync_copy(normed_hbm.at[pl.ds(wid * TPW, TPW)], buf)
    s0 = pltpu.async_copy(buf, xs_hbm.at[idx_v.at[0]], sem0)
    s1 = pltpu.async_copy(buf, xs_hbm.at[idx_v.at[1]], sem1)
    s0.wait()
    s1.wait()


@functools.partial(
    pl.kernel,
    out_type=(jax.ShapeDtypeStruct((T, NF * H), jnp.float32),
              jax.ShapeDtypeStruct((T, NF * H), jnp.float32)),
    mesh=plsc.VectorSubcoreMesh(core_axis_name="c", subcore_axis_name="s"),
    scratch_types=[
        pltpu.VMEM((K, TPW), jnp.int32),
        pltpu.VMEM((TPW // 2, NF * H), jnp.float32),
        pltpu.VMEM((TPW // 2, NF * H), jnp.float32),
        pltpu.SemaphoreType.DMA,
        pltpu.SemaphoreType.DMA,
    ],
)
def _combine_gather(y_hbm, idx3_hbm, y0_hbm, y1_hbm, idx_v, b0, b1,
                    sem0, sem1):
    wid = lax.axis_index("s") * 2 + lax.axis_index("c")
    base = wid * TPW
    hw = TPW // 2
    pltpu.sync_copy(idx3_hbm.at[wid], idx_v)
    for c in range(2):
        g0 = pltpu.async_copy(y_hbm.at[idx_v.at[0, pl.ds(c * hw, hw)]],
                              b0, sem0)
        g1 = pltpu.async_copy(y_hbm.at[idx_v.at[1, pl.ds(c * hw, hw)]],
                              b1, sem1)
        g0.wait()
        pltpu.sync_copy(b0, y0_hbm.at[pl.ds(base + c * hw, hw)])
        g1.wait()
        pltpu.sync_copy(b1, y1_hbm.at[pl.ds(base + c * hw, hw)])


def _ffn_body(te_ref, tv_ref, xs_ref, wg_ref, wu_ref, wd_ref, y_ref):
    i = pl.program_id(1)

    @pl.when(tv_ref[i] == 1)
    def _():
        x = xs_ref[...]
        g = jnp.dot(x, wg_ref[0], preferred_element_type=jnp.float32)
        u = jnp.dot(x, wu_ref[0], preferred_element_type=jnp.float32)
        h = (g * jax.nn.sigmoid(g)) * u
        y_ref[...] = jnp.dot(h, wd_ref[0], preferred_element_type=jnp.float32)


def _grouped_ffn(xs, W_gate, W_up, W_down, tile_expert, tile_valid):
    grid_spec = pltpu.PrefetchScalarGridSpec(
        num_scalar_prefetch=2,
        grid=(NF, NT),
        in_specs=[
            pl.BlockSpec((TILE, H), lambda j, i, te, tv: (i, 0)),
            pl.BlockSpec((1, H, FC), lambda j, i, te, tv: (te[i], 0, j)),
            pl.BlockSpec((1, H, FC), lambda j, i, te, tv: (te[i], 0, j)),
            pl.BlockSpec((1, FC, H), lambda j, i, te, tv: (te[i], j, 0)),
        ],
        out_specs=pl.BlockSpec((TILE, H), lambda j, i, te, tv: (i, j)),
    )
    return pl.pallas_call(
        _ffn_body,
        grid_spec=grid_spec,
        out_shape=jax.ShapeDtypeStruct((A_PAD, NF * H), jnp.float32),
        compiler_params=pltpu.CompilerParams(
            dimension_semantics=("arbitrary", "arbitrary")),
    )(tile_expert, tile_valid, xs, W_gate, W_up, W_down)


def _combine_body(x_ref, y0_ref, y1_ref, w_ref, out_ref):
    w = w_ref[...]
    y0 = y0_ref[:, :H] + y0_ref[:, H:]
    y1 = y1_ref[:, :H] + y1_ref[:, H:]
    out_ref[...] = x_ref[...] + w[:, 0:1] * y0 + w[:, 1:2] * y1


def _combine(x, y0, y1, wsel):
    return pl.pallas_call(
        _combine_body,
        grid=(T // RT,),
        in_specs=[
            pl.BlockSpec((RT, H), lambda i: (i, 0)),
            pl.BlockSpec((RT, NF * H), lambda i: (i, 0)),
            pl.BlockSpec((RT, NF * H), lambda i: (i, 0)),
            pl.BlockSpec((RT, K), lambda i: (i, 0)),
        ],
        out_specs=pl.BlockSpec((RT, H), lambda i: (i, 0)),
        out_shape=jax.ShapeDtypeStruct((T, H), jnp.float32),
    )(x, y0, y1, wsel)


def kernel(hidden_states, gate_W, W_gate, W_up, W_down, ln_g, ln_b):
    B, S, _ = hidden_states.shape
    x = hidden_states.reshape(T, H)

    normed, isel, wsel, _, _, aux = _router(x, gate_W, ln_g, ln_b)

    e0_2d = isel[:, 0].reshape(GR, GC)
    e1_2d = isel[:, 1].reshape(GR, GC)
    p0_2d, p1_2d, te_2d, tv_2d = _glue(e0_2d, e1_2d)

    idx3 = jnp.stack([p0_2d.reshape(NW, TPW), p1_2d.reshape(NW, TPW)],
                     axis=1)
    tile_expert = te_2d.reshape(NT)
    tile_valid = tv_2d.reshape(NT)

    xs = _dispatch_scatter(normed, idx3)

    y = _grouped_ffn(xs, W_gate, W_up, W_down, tile_expert, tile_valid)

    y0, y1 = _combine_gather(y, idx3)
    out = _combine(x, y0, y1, wsel)

    return out.reshape(B, S, H), aux[0, 0]
